# baseline (speedup 1.0000x reference)
import jax
import jax.numpy as jnp
from jax import lax
from jax.experimental import pallas as pl

_N_CLASSES = 1000
_BLOCK_I = 64


def _onehot_body(idx_ref, out_ref):
    idx = idx_ref[...]
    classes = lax.broadcasted_iota(jnp.int32, out_ref.shape, 2)
    out_ref[...] = (idx == classes).astype(jnp.float32)


def kernel(inputs):
    n, m = inputs.shape
    grid = (n // _BLOCK_I,)
    idx3 = inputs.reshape(n, m, 1)
    return pl.pallas_call(
        _onehot_body,
        grid=grid,
        in_specs=[pl.BlockSpec((_BLOCK_I, m, 1), lambda i: (i, 0, 0))],
        out_specs=pl.BlockSpec((_BLOCK_I, m, _N_CLASSES), lambda i: (i, 0, 0)),
        out_shape=jax.ShapeDtypeStruct((n, m, _N_CLASSES), jnp.float32),
    )(idx3)


# speedup vs baseline: 1.0150x; 1.0150x over previous
import jax
import jax.numpy as jnp
from jax import lax
from jax.experimental import pallas as pl

_N_CLASSES = 1000
_BLOCK_I = 256


def _onehot_body(idx_ref, out_ref):
    idx = idx_ref[...]
    classes = lax.broadcasted_iota(jnp.int32, out_ref.shape, 2)
    out_ref[...] = (idx == classes).astype(jnp.float32)


def kernel(inputs):
    n, m = inputs.shape
    grid = (n // _BLOCK_I,)
    idx3 = inputs.reshape(n, m, 1)
    return pl.pallas_call(
        _onehot_body,
        grid=grid,
        in_specs=[pl.BlockSpec((_BLOCK_I, m, 1), lambda i: (i, 0, 0))],
        out_specs=pl.BlockSpec((_BLOCK_I, m, _N_CLASSES), lambda i: (i, 0, 0)),
        out_shape=jax.ShapeDtypeStruct((n, m, _N_CLASSES), jnp.float32),
    )(idx3)


# speedup vs baseline: 1.0957x; 1.0795x over previous
import jax
import jax.numpy as jnp
from jax import lax
from jax.experimental import pallas as pl
from jax.experimental.pallas import tpu as pltpu

_N_CLASSES = 1000
_BLOCK_I = 64
_NBUF = 4


def _onehot_body(idx_ref, out_ref, buf_ref, sem):
    i = pl.program_id(0)
    n_steps = pl.num_programs(0)
    idx = idx_ref[...]
    classes = lax.broadcasted_iota(
        jnp.int32, (_BLOCK_I, idx.shape[1], _N_CLASSES), 2)
    vals = (idx[:, :, None] == classes).astype(jnp.float32)

    for k in range(_NBUF):

        @pl.when(jnp.logical_and(i % _NBUF == k, i >= _NBUF))
        def _wait():
            pltpu.make_async_copy(
                buf_ref.at[k],
                out_ref.at[pl.ds((i - _NBUF) * _BLOCK_I, _BLOCK_I)],
                sem.at[k],
            ).wait()

        @pl.when(i % _NBUF == k)
        def _issue():
            buf_ref[k] = vals
            pltpu.make_async_copy(
                buf_ref.at[k],
                out_ref.at[pl.ds(i * _BLOCK_I, _BLOCK_I)],
                sem.at[k],
            ).start()

    @pl.when(i == n_steps - 1)
    def _drain():
        for k in range(_NBUF):
            step = n_steps - _NBUF + k
            pltpu.make_async_copy(
                buf_ref.at[(n_steps - _NBUF + k) % _NBUF],
                out_ref.at[pl.ds(step * _BLOCK_I, _BLOCK_I)],
                sem.at[(n_steps - _NBUF + k) % _NBUF],
            ).wait()


def kernel(inputs):
    n, m = inputs.shape
    grid = (n // _BLOCK_I,)
    return pl.pallas_call(
        _onehot_body,
        grid=grid,
        in_specs=[pl.BlockSpec((_BLOCK_I, m), lambda i: (i, 0))],
        out_specs=pl.BlockSpec(memory_space=pl.ANY),
        out_shape=jax.ShapeDtypeStruct((n, m, _N_CLASSES), jnp.float32),
        scratch_shapes=[
            pltpu.VMEM((_NBUF, _BLOCK_I, m, _N_CLASSES), jnp.float32),
            pltpu.SemaphoreType.DMA((_NBUF,)),
        ],
    )(inputs)


# speedup vs baseline: 4.9052x; 4.4768x over previous
import jax
import jax.numpy as jnp
from jax import lax
from jax.experimental import pallas as pl

_N_CLASSES = 1000
_BLOCK_K = 40


def _onehot_body(idx_ref, out_ref):
    q = pl.program_id(0)
    idx = idx_ref[...]
    m, bk, n = out_ref.shape
    classes = lax.broadcasted_iota(jnp.int32, (m, bk, n), 1) + q * bk
    out_ref[...] = (idx[:, None, :] == classes).astype(jnp.float32)


def kernel(inputs):
    n, m = inputs.shape
    idx_t = inputs.T
    out_t = pl.pallas_call(
        _onehot_body,
        grid=(_N_CLASSES // _BLOCK_K,),
        in_specs=[pl.BlockSpec((m, n), lambda q: (0, 0))],
        out_specs=pl.BlockSpec((m, _BLOCK_K, n), lambda q: (0, q, 0)),
        out_shape=jax.ShapeDtypeStruct((m, _N_CLASSES, n), jnp.float32),
    )(idx_t)
    return jnp.transpose(out_t, (2, 0, 1))
